# baseline (speedup 1.0000x reference)
import functools

import jax
import jax.numpy as jnp
from jax import lax
from jax.experimental import pallas as pl
from jax.experimental.pallas import tpu as pltpu
from jax.experimental.pallas import tpu_sc as plsc

_B, _H, _M, _R, _K = 64, 4, 128, 4096, 32
_NB = 8
_NH = _NB * _H
_T = _B * _H
_NEG = -1e30
_BIG = 2**30


def _score_body(keys_ref, sraw_ref, mem_ref, wgt_ref, idx_ref):
    s_parts = []
    for nb in range(_NB):
        keys = keys_ref[nb]
        sraw = sraw_ref[nb]
        mem = mem_ref[nb]

        knorm = jnp.sqrt(jnp.sum(keys * keys, axis=1, keepdims=True))
        kn = keys / jnp.maximum(knorm, 1e-12)

        rs = jnp.sum(mem * mem, axis=1, keepdims=True)
        inv = (1.0 / jnp.sqrt(rs.reshape(_R // _M, _M))).reshape(_R, 1)
        sm = mem * inv

        cos = jax.lax.dot_general(
            kn, sm, (((1,), (1,)), ((), ())),
            preferred_element_type=jnp.float32)
        strength = (jnp.maximum(sraw, 0.0)
                    + jnp.log1p(jnp.exp(-jnp.abs(sraw))))
        s_parts.append(strength * cos)

    s = jnp.concatenate(s_parts, axis=0)

    iota = lax.broadcasted_iota(jnp.int32, (_NH, _R), 1)
    kiota = lax.broadcasted_iota(jnp.int32, (_NH, _K), 1)
    vals0 = jnp.full((_NH, _K), _NEG, jnp.float32)
    idxs0 = jnp.zeros((_NH, _K), jnp.int32)

    def step(i, carry):
        w, vals, idxs = carry
        m = jnp.max(w, axis=1, keepdims=True)
        t = jnp.where(w == m, iota, _BIG)
        mi = jnp.min(t, axis=1, keepdims=True)
        vals = jnp.where(kiota == i, m, vals)
        idxs = jnp.where(kiota == i, mi, idxs)
        return jnp.where(t == mi, _NEG, w), vals, idxs

    _, vals, idxs = lax.fori_loop(0, _K, step, (s, vals0, idxs0))

    mx = jnp.max(vals, axis=1, keepdims=True)
    e = jnp.exp(vals - mx)
    wgt_ref[...] = e / jnp.sum(e, axis=1, keepdims=True)

    riota = lax.broadcasted_iota(jnp.int32, (_NH, _K), 0)
    base = (pl.program_id(0) * _NB + riota // _H) * _R
    idx_ref[...] = idxs + base


def _gather_body(mem_hbm, idx_hbm, wgt_hbm, out_hbm,
                 idx_v0, idx_v1, rows_v0, rows_v1, wgt_v, out_v, sem):
    info = plsc.get_sparse_core_info()
    nc = info.num_cores
    wid = lax.axis_index("s") * nc + lax.axis_index("c")
    ntasks = _T // (nc * info.num_subcores)
    rows_w = ntasks * _K
    base = wid * rows_w
    pltpu.sync_copy(idx_hbm.at[pl.ds(base, rows_w // 2)], idx_v0)
    pltpu.sync_copy(idx_hbm.at[pl.ds(base + rows_w // 2, rows_w // 2)],
                    idx_v1)
    pltpu.sync_copy(wgt_hbm.at[pl.ds(base, rows_w)], wgt_v)
    cp0 = pltpu.async_copy(mem_hbm.at[idx_v0], rows_v0, sem)
    cp1 = pltpu.async_copy(mem_hbm.at[idx_v1], rows_v1, sem)
    cp0.wait()
    cp1.wait()

    for i in range(ntasks):
        rv = rows_v0 if i < ntasks // 2 else rows_v1
        roff = (i % (ntasks // 2)) * _K
        wv = [wgt_v[pl.ds(i * _K + g * 16, 16)] for g in range(_K // 16)]
        for c in range(_M // 16):
            acc = jnp.zeros((16,), jnp.float32)
            for j in range(_K):
                wj = wv[j // 16][j % 16]
                acc = acc + wj * rv[roff + j, pl.ds(c * 16, 16)]
            out_v[pl.ds(i * _M + c * 16, 16)] = acc
    pltpu.sync_copy(out_v, out_hbm.at[pl.ds(wid * ntasks * _M, ntasks * _M)])


def kernel(read_inputs, mem_state):
    keys = read_inputs[:, :_H * _M].reshape(_B, _H, _M)
    sraw = read_inputs[:, _H * _M:].reshape(_B, _H, 1)
    wgt, idx = pl.pallas_call(
        _score_body,
        grid=(_B // _NB,),
        in_specs=[
            pl.BlockSpec((_NB, _H, _M), lambda b: (b, 0, 0)),
            pl.BlockSpec((_NB, _H, 1), lambda b: (b, 0, 0)),
            pl.BlockSpec((_NB, _R, _M), lambda b: (b, 0, 0)),
        ],
        out_specs=[
            pl.BlockSpec((_NH, _K), lambda b: (b, 0)),
            pl.BlockSpec((_NH, _K), lambda b: (b, 0)),
        ],
        out_shape=[
            jax.ShapeDtypeStruct((_T, _K), jnp.float32),
            jax.ShapeDtypeStruct((_T, _K), jnp.int32),
        ],
    )(keys, sraw, mem_state)

    mesh = plsc.VectorSubcoreMesh(core_axis_name="c", subcore_axis_name="s")
    gather_sum = pl.kernel(
        _gather_body, mesh=mesh,
        out_type=jax.ShapeDtypeStruct((_T * _M,), jnp.float32),
        scratch_types=[
            pltpu.VMEM((128,), jnp.int32),
            pltpu.VMEM((128,), jnp.int32),
            pltpu.VMEM((128, _M), jnp.float32),
            pltpu.VMEM((128, _M), jnp.float32),
            pltpu.VMEM((256,), jnp.float32),
            pltpu.VMEM((8 * _M,), jnp.float32),
            pltpu.SemaphoreType.DMA,
        ],
    )
    out = gather_sum(mem_state.reshape(_B * _R, _M),
                     idx.reshape(_T * _K), wgt.reshape(_T * _K))
    return out.reshape(_B, _H * _M)


# speedup vs baseline: 1.1190x; 1.1190x over previous
import functools

import jax
import jax.numpy as jnp
from jax import lax
from jax.experimental import pallas as pl
from jax.experimental.pallas import tpu as pltpu
from jax.experimental.pallas import tpu_sc as plsc

_B, _H, _M, _R, _K = 64, 4, 128, 4096, 32
_NB = 8
_NH = _NB * _H
_T = _B * _H
_NEG = -1e30
_BIG = 2**30


def _score_body(keys_ref, sraw_ref, mem_ref, wgt_ref, idx_ref):
    s_parts = []
    for nb in range(_NB):
        keys = keys_ref[nb]
        sraw = sraw_ref[nb]
        mem = mem_ref[nb]

        knorm = jnp.sqrt(jnp.sum(keys * keys, axis=1, keepdims=True))
        kn = keys / jnp.maximum(knorm, 1e-12)

        rs = jnp.sum(mem * mem, axis=1, keepdims=True)
        inv = (1.0 / jnp.sqrt(rs.reshape(_R // _M, _M))).reshape(_R, 1)
        sm = mem * inv

        cos = jax.lax.dot_general(
            kn, sm, (((1,), (1,)), ((), ())),
            preferred_element_type=jnp.float32)
        strength = (jnp.maximum(sraw, 0.0)
                    + jnp.log1p(jnp.exp(-jnp.abs(sraw))))
        s_parts.append(strength * cos)

    s = jnp.concatenate(s_parts, axis=0)

    iota = lax.broadcasted_iota(jnp.int32, (_NH, _R), 1)
    kiota = lax.broadcasted_iota(jnp.int32, (_NH, _K), 1)
    vals0 = jnp.full((_NH, _K), _NEG, jnp.float32)
    idxs0 = jnp.zeros((_NH, _K), jnp.int32)

    def step(i, carry):
        w, vals, idxs = carry
        m = jnp.max(w, axis=1, keepdims=True)
        t = jnp.where(w == m, iota, _BIG)
        mi = jnp.min(t, axis=1, keepdims=True)
        vals = jnp.where(kiota == i, m, vals)
        idxs = jnp.where(kiota == i, mi, idxs)
        return jnp.where(t == mi, _NEG, w), vals, idxs

    _, vals, idxs = lax.fori_loop(0, _K, step, (s, vals0, idxs0),
                                  unroll=4)

    mx = jnp.max(vals, axis=1, keepdims=True)
    e = jnp.exp(vals - mx)
    wgt_ref[...] = e / jnp.sum(e, axis=1, keepdims=True)

    riota = lax.broadcasted_iota(jnp.int32, (_NH, _K), 0)
    base = (pl.program_id(0) * _NB + riota // _H) * _R
    idx_ref[...] = idxs + base


def _gather_body(mem_hbm, idx_hbm, wgt_hbm, out_hbm,
                 idx_v0, idx_v1, rows_v0, rows_v1, wgt_v, out_v, sem):
    info = plsc.get_sparse_core_info()
    nc = info.num_cores
    wid = lax.axis_index("s") * nc + lax.axis_index("c")
    ntasks = _T // (nc * info.num_subcores)
    rows_w = ntasks * _K
    base = wid * rows_w
    pltpu.sync_copy(idx_hbm.at[pl.ds(base, rows_w // 2)], idx_v0)
    pltpu.sync_copy(idx_hbm.at[pl.ds(base + rows_w // 2, rows_w // 2)],
                    idx_v1)
    pltpu.sync_copy(wgt_hbm.at[pl.ds(base, rows_w)], wgt_v)
    cp0 = pltpu.async_copy(mem_hbm.at[idx_v0], rows_v0, sem)
    cp1 = pltpu.async_copy(mem_hbm.at[idx_v1], rows_v1, sem)
    cp0.wait()
    cp1.wait()

    for i in range(ntasks):
        rv = rows_v0 if i < ntasks // 2 else rows_v1
        roff = (i % (ntasks // 2)) * _K
        wv = [wgt_v[pl.ds(i * _K + g * 16, 16)] for g in range(_K // 16)]
        for c in range(_M // 16):
            acc = jnp.zeros((16,), jnp.float32)
            for j in range(_K):
                wj = wv[j // 16][j % 16]
                acc = acc + wj * rv[roff + j, pl.ds(c * 16, 16)]
            out_v[pl.ds(i * _M + c * 16, 16)] = acc
    pltpu.sync_copy(out_v, out_hbm.at[pl.ds(wid * ntasks * _M, ntasks * _M)])


def kernel(read_inputs, mem_state):
    keys = read_inputs[:, :_H * _M].reshape(_B, _H, _M)
    sraw = read_inputs[:, _H * _M:].reshape(_B, _H, 1)
    wgt, idx = pl.pallas_call(
        _score_body,
        grid=(_B // _NB,),
        in_specs=[
            pl.BlockSpec((_NB, _H, _M), lambda b: (b, 0, 0)),
            pl.BlockSpec((_NB, _H, 1), lambda b: (b, 0, 0)),
            pl.BlockSpec((_NB, _R, _M), lambda b: (b, 0, 0)),
        ],
        out_specs=[
            pl.BlockSpec((_NH, _K), lambda b: (b, 0)),
            pl.BlockSpec((_NH, _K), lambda b: (b, 0)),
        ],
        out_shape=[
            jax.ShapeDtypeStruct((_T, _K), jnp.float32),
            jax.ShapeDtypeStruct((_T, _K), jnp.int32),
        ],
    )(keys, sraw, mem_state)

    mesh = plsc.VectorSubcoreMesh(core_axis_name="c", subcore_axis_name="s")
    gather_sum = pl.kernel(
        _gather_body, mesh=mesh,
        out_type=jax.ShapeDtypeStruct((_T * _M,), jnp.float32),
        scratch_types=[
            pltpu.VMEM((128,), jnp.int32),
            pltpu.VMEM((128,), jnp.int32),
            pltpu.VMEM((128, _M), jnp.float32),
            pltpu.VMEM((128, _M), jnp.float32),
            pltpu.VMEM((256,), jnp.float32),
            pltpu.VMEM((8 * _M,), jnp.float32),
            pltpu.SemaphoreType.DMA,
        ],
    )
    out = gather_sum(mem_state.reshape(_B * _R, _M),
                     idx.reshape(_T * _K), wgt.reshape(_T * _K))
    return out.reshape(_B, _H * _M)


# speedup vs baseline: 1.1411x; 1.0197x over previous
import functools

import jax
import jax.numpy as jnp
from jax import lax
from jax.experimental import pallas as pl
from jax.experimental.pallas import tpu as pltpu
from jax.experimental.pallas import tpu_sc as plsc

_B, _H, _M, _R, _K = 64, 4, 128, 4096, 32
_NB = 8
_NH = _NB * _H
_T = _B * _H
_NEG = -1e30
_BIG = 2**30


def _score_body(keys_ref, sraw_ref, mem_ref, wgt_ref, idx_ref):
    s_parts = []
    for nb in range(_NB):
        keys = keys_ref[nb]
        sraw = sraw_ref[nb]
        mem = mem_ref[nb]

        knorm = jnp.sqrt(jnp.sum(keys * keys, axis=1, keepdims=True))
        kn = keys / jnp.maximum(knorm, 1e-12)

        rs = jnp.sum(mem * mem, axis=1, keepdims=True)
        inv = (1.0 / jnp.sqrt(rs.reshape(_R // _M, _M))).reshape(_R, 1)
        sm = mem * inv

        cos = jax.lax.dot_general(
            kn, sm, (((1,), (1,)), ((), ())),
            preferred_element_type=jnp.float32)
        strength = (jnp.maximum(sraw, 0.0)
                    + jnp.log1p(jnp.exp(-jnp.abs(sraw))))
        s_parts.append(strength * cos)

    s = jnp.concatenate(s_parts, axis=0)

    iota = lax.broadcasted_iota(jnp.int32, (_NH, _R), 1)
    kiota = lax.broadcasted_iota(jnp.int32, (_NH, _K), 1)
    vals0 = jnp.full((_NH, _K), _NEG, jnp.float32)
    idxs0 = jnp.zeros((_NH, _K), jnp.int32)

    def step(i, carry):
        w, vals, idxs = carry
        m = jnp.max(w, axis=1, keepdims=True)
        t = jnp.where(w == m, iota, _BIG)
        mi = jnp.min(t, axis=1, keepdims=True)
        vals = jnp.where(kiota == i, m, vals)
        idxs = jnp.where(kiota == i, mi, idxs)
        return jnp.where(t == mi, _NEG, w), vals, idxs

    _, vals, idxs = lax.fori_loop(0, _K, step, (s, vals0, idxs0),
                                  unroll=8)

    mx = jnp.max(vals, axis=1, keepdims=True)
    e = jnp.exp(vals - mx)
    wgt_ref[...] = e / jnp.sum(e, axis=1, keepdims=True)

    riota = lax.broadcasted_iota(jnp.int32, (_NH, _K), 0)
    base = (pl.program_id(0) * _NB + riota // _H) * _R
    idx_ref[...] = idxs + base


def _gather_body(mem_hbm, idx_hbm, wgt_hbm, out_hbm,
                 idx_v0, idx_v1, rows_v0, rows_v1, wgt_v, out_v, sem):
    info = plsc.get_sparse_core_info()
    nc = info.num_cores
    wid = lax.axis_index("s") * nc + lax.axis_index("c")
    ntasks = _T // (nc * info.num_subcores)
    rows_w = ntasks * _K
    base = wid * rows_w
    pltpu.sync_copy(idx_hbm.at[pl.ds(base, rows_w // 2)], idx_v0)
    pltpu.sync_copy(idx_hbm.at[pl.ds(base + rows_w // 2, rows_w // 2)],
                    idx_v1)
    pltpu.sync_copy(wgt_hbm.at[pl.ds(base, rows_w)], wgt_v)
    cp0 = pltpu.async_copy(mem_hbm.at[idx_v0], rows_v0, sem)
    cp1 = pltpu.async_copy(mem_hbm.at[idx_v1], rows_v1, sem)
    cp0.wait()
    cp1.wait()

    for i in range(ntasks):
        rv = rows_v0 if i < ntasks // 2 else rows_v1
        roff = (i % (ntasks // 2)) * _K
        wv = [wgt_v[pl.ds(i * _K + g * 16, 16)] for g in range(_K // 16)]
        for c in range(_M // 16):
            acc = jnp.zeros((16,), jnp.float32)
            for j in range(_K):
                wj = wv[j // 16][j % 16]
                acc = acc + wj * rv[roff + j, pl.ds(c * 16, 16)]
            out_v[pl.ds(i * _M + c * 16, 16)] = acc
    pltpu.sync_copy(out_v, out_hbm.at[pl.ds(wid * ntasks * _M, ntasks * _M)])


def kernel(read_inputs, mem_state):
    keys = read_inputs[:, :_H * _M].reshape(_B, _H, _M)
    sraw = read_inputs[:, _H * _M:].reshape(_B, _H, 1)
    wgt, idx = pl.pallas_call(
        _score_body,
        grid=(_B // _NB,),
        in_specs=[
            pl.BlockSpec((_NB, _H, _M), lambda b: (b, 0, 0)),
            pl.BlockSpec((_NB, _H, 1), lambda b: (b, 0, 0)),
            pl.BlockSpec((_NB, _R, _M), lambda b: (b, 0, 0)),
        ],
        out_specs=[
            pl.BlockSpec((_NH, _K), lambda b: (b, 0)),
            pl.BlockSpec((_NH, _K), lambda b: (b, 0)),
        ],
        out_shape=[
            jax.ShapeDtypeStruct((_T, _K), jnp.float32),
            jax.ShapeDtypeStruct((_T, _K), jnp.int32),
        ],
    )(keys, sraw, mem_state)

    mesh = plsc.VectorSubcoreMesh(core_axis_name="c", subcore_axis_name="s")
    gather_sum = pl.kernel(
        _gather_body, mesh=mesh,
        out_type=jax.ShapeDtypeStruct((_T * _M,), jnp.float32),
        scratch_types=[
            pltpu.VMEM((128,), jnp.int32),
            pltpu.VMEM((128,), jnp.int32),
            pltpu.VMEM((128, _M), jnp.float32),
            pltpu.VMEM((128, _M), jnp.float32),
            pltpu.VMEM((256,), jnp.float32),
            pltpu.VMEM((8 * _M,), jnp.float32),
            pltpu.SemaphoreType.DMA,
        ],
    )
    out = gather_sum(mem_state.reshape(_B * _R, _M),
                     idx.reshape(_T * _K), wgt.reshape(_T * _K))
    return out.reshape(_B, _H * _M)


# speedup vs baseline: 1.1571x; 1.0141x over previous
import functools

import jax
import jax.numpy as jnp
from jax import lax
from jax.experimental import pallas as pl
from jax.experimental.pallas import tpu as pltpu
from jax.experimental.pallas import tpu_sc as plsc

_B, _H, _M, _R, _K = 64, 4, 128, 4096, 32
_NB = 8
_NH = _NB * _H
_T = _B * _H
_NEG = -1e30
_BIG = 2**30


def _score_body(keys_ref, sraw_ref, mem_ref, wgt_ref, idx_ref):
    s_parts = []
    for nb in range(_NB):
        keys = keys_ref[nb]
        sraw = sraw_ref[nb]
        mem = mem_ref[nb]

        knorm = jnp.sqrt(jnp.sum(keys * keys, axis=1, keepdims=True))
        kn = keys / jnp.maximum(knorm, 1e-12)

        rs = jnp.sum(mem * mem, axis=1, keepdims=True)
        inv = (1.0 / jnp.sqrt(rs.reshape(_R // _M, _M))).reshape(_R, 1)
        sm = mem * inv

        cos = jax.lax.dot_general(
            kn, sm, (((1,), (1,)), ((), ())),
            preferred_element_type=jnp.float32)
        strength = (jnp.maximum(sraw, 0.0)
                    + jnp.log1p(jnp.exp(-jnp.abs(sraw))))
        s_parts.append(strength * cos)

    s = jnp.concatenate(s_parts, axis=0)

    iota = lax.broadcasted_iota(jnp.int32, (_NH, _R), 1)
    kiota = lax.broadcasted_iota(jnp.int32, (_NH, _K), 1)
    vals0 = jnp.full((_NH, _K), _NEG, jnp.float32)
    idxs0 = jnp.zeros((_NH, _K), jnp.int32)

    def step(i, carry):
        w, vals, idxs = carry
        m = jnp.max(w, axis=1, keepdims=True)
        t = jnp.where(w == m, iota, _BIG)
        mi = jnp.min(t, axis=1, keepdims=True)
        vals = jnp.where(kiota == i, m, vals)
        idxs = jnp.where(kiota == i, mi, idxs)
        return jnp.where(t == mi, _NEG, w), vals, idxs

    _, vals, idxs = lax.fori_loop(0, _K, step, (s, vals0, idxs0),
                                  unroll=16)

    mx = jnp.max(vals, axis=1, keepdims=True)
    e = jnp.exp(vals - mx)
    wgt_ref[...] = e / jnp.sum(e, axis=1, keepdims=True)

    riota = lax.broadcasted_iota(jnp.int32, (_NH, _K), 0)
    base = (pl.program_id(0) * _NB + riota // _H) * _R
    idx_ref[...] = idxs + base


def _gather_body(mem_hbm, idx_hbm, wgt_hbm, out_hbm,
                 idx_v0, idx_v1, rows_v0, rows_v1, wgt_v, out_v, sem):
    info = plsc.get_sparse_core_info()
    nc = info.num_cores
    wid = lax.axis_index("s") * nc + lax.axis_index("c")
    ntasks = _T // (nc * info.num_subcores)
    rows_w = ntasks * _K
    base = wid * rows_w
    pltpu.sync_copy(idx_hbm.at[pl.ds(base, rows_w // 2)], idx_v0)
    pltpu.sync_copy(idx_hbm.at[pl.ds(base + rows_w // 2, rows_w // 2)],
                    idx_v1)
    pltpu.sync_copy(wgt_hbm.at[pl.ds(base, rows_w)], wgt_v)
    cp0 = pltpu.async_copy(mem_hbm.at[idx_v0], rows_v0, sem)
    cp1 = pltpu.async_copy(mem_hbm.at[idx_v1], rows_v1, sem)
    cp0.wait()
    cp1.wait()

    for i in range(ntasks):
        rv = rows_v0 if i < ntasks // 2 else rows_v1
        roff = (i % (ntasks // 2)) * _K
        wv = [wgt_v[pl.ds(i * _K + g * 16, 16)] for g in range(_K // 16)]
        for c in range(_M // 16):
            acc = jnp.zeros((16,), jnp.float32)
            for j in range(_K):
                wj = wv[j // 16][j % 16]
                acc = acc + wj * rv[roff + j, pl.ds(c * 16, 16)]
            out_v[pl.ds(i * _M + c * 16, 16)] = acc
    pltpu.sync_copy(out_v, out_hbm.at[pl.ds(wid * ntasks * _M, ntasks * _M)])


def kernel(read_inputs, mem_state):
    keys = read_inputs[:, :_H * _M].reshape(_B, _H, _M)
    sraw = read_inputs[:, _H * _M:].reshape(_B, _H, 1)
    wgt, idx = pl.pallas_call(
        _score_body,
        grid=(_B // _NB,),
        in_specs=[
            pl.BlockSpec((_NB, _H, _M), lambda b: (b, 0, 0)),
            pl.BlockSpec((_NB, _H, 1), lambda b: (b, 0, 0)),
            pl.BlockSpec((_NB, _R, _M), lambda b: (b, 0, 0)),
        ],
        out_specs=[
            pl.BlockSpec((_NH, _K), lambda b: (b, 0)),
            pl.BlockSpec((_NH, _K), lambda b: (b, 0)),
        ],
        out_shape=[
            jax.ShapeDtypeStruct((_T, _K), jnp.float32),
            jax.ShapeDtypeStruct((_T, _K), jnp.int32),
        ],
    )(keys, sraw, mem_state)

    mesh = plsc.VectorSubcoreMesh(core_axis_name="c", subcore_axis_name="s")
    gather_sum = pl.kernel(
        _gather_body, mesh=mesh,
        out_type=jax.ShapeDtypeStruct((_T * _M,), jnp.float32),
        scratch_types=[
            pltpu.VMEM((128,), jnp.int32),
            pltpu.VMEM((128,), jnp.int32),
            pltpu.VMEM((128, _M), jnp.float32),
            pltpu.VMEM((128, _M), jnp.float32),
            pltpu.VMEM((256,), jnp.float32),
            pltpu.VMEM((8 * _M,), jnp.float32),
            pltpu.SemaphoreType.DMA,
        ],
    )
    out = gather_sum(mem_state.reshape(_B * _R, _M),
                     idx.reshape(_T * _K), wgt.reshape(_T * _K))
    return out.reshape(_B, _H * _M)


# speedup vs baseline: 1.1812x; 1.0208x over previous
import functools

import jax
import jax.numpy as jnp
from jax import lax
from jax.experimental import pallas as pl
from jax.experimental.pallas import tpu as pltpu
from jax.experimental.pallas import tpu_sc as plsc

_B, _H, _M, _R, _K = 64, 4, 128, 4096, 32
_NB = 8
_NH = _NB * _H
_T = _B * _H
_NEG = -1e30
_BIG = 2**30


def _score_body(keys_ref, sraw_ref, mem_ref, wgt_ref, idx_ref):
    s_parts = []
    for nb in range(_NB):
        keys = keys_ref[nb]
        sraw = sraw_ref[nb]
        mem = mem_ref[nb]

        knorm = jnp.sqrt(jnp.sum(keys * keys, axis=1, keepdims=True))
        kn = keys / jnp.maximum(knorm, 1e-12)

        rs = jnp.sum(mem * mem, axis=1, keepdims=True)
        inv = (1.0 / jnp.sqrt(rs.reshape(_R // _M, _M))).reshape(_R, 1)
        sm = mem * inv

        cos = jax.lax.dot_general(
            kn, sm, (((1,), (1,)), ((), ())),
            preferred_element_type=jnp.float32)
        strength = (jnp.maximum(sraw, 0.0)
                    + jnp.log1p(jnp.exp(-jnp.abs(sraw))))
        s_parts.append(strength * cos)

    s = jnp.concatenate(s_parts, axis=0)

    iota = lax.broadcasted_iota(jnp.int32, (_NH, _R), 1)
    kiota = lax.broadcasted_iota(jnp.int32, (_NH, _K), 1)
    vals0 = jnp.full((_NH, _K), _NEG, jnp.float32)
    idxs0 = jnp.zeros((_NH, _K), jnp.int32)

    def step(i, carry):
        w, vals, idxs = carry
        m = jnp.max(w, axis=1, keepdims=True)
        t = jnp.where(w == m, iota, _BIG)
        mi = jnp.min(t, axis=1, keepdims=True)
        vals = jnp.where(kiota == i, m, vals)
        idxs = jnp.where(kiota == i, mi, idxs)
        return jnp.where(t == mi, _NEG, w), vals, idxs

    _, vals, idxs = lax.fori_loop(0, _K, step, (s, vals0, idxs0),
                                  unroll=32)

    mx = jnp.max(vals, axis=1, keepdims=True)
    e = jnp.exp(vals - mx)
    wgt_ref[...] = e / jnp.sum(e, axis=1, keepdims=True)

    riota = lax.broadcasted_iota(jnp.int32, (_NH, _K), 0)
    base = (pl.program_id(0) * _NB + riota // _H) * _R
    idx_ref[...] = idxs + base


def _gather_body(mem_hbm, idx_hbm, wgt_hbm, out_hbm,
                 idx_v0, idx_v1, rows_v0, rows_v1, wgt_v, out_v, sem):
    info = plsc.get_sparse_core_info()
    nc = info.num_cores
    wid = lax.axis_index("s") * nc + lax.axis_index("c")
    ntasks = _T // (nc * info.num_subcores)
    rows_w = ntasks * _K
    base = wid * rows_w
    pltpu.sync_copy(idx_hbm.at[pl.ds(base, rows_w // 2)], idx_v0)
    pltpu.sync_copy(idx_hbm.at[pl.ds(base + rows_w // 2, rows_w // 2)],
                    idx_v1)
    pltpu.sync_copy(wgt_hbm.at[pl.ds(base, rows_w)], wgt_v)
    cp0 = pltpu.async_copy(mem_hbm.at[idx_v0], rows_v0, sem)
    cp1 = pltpu.async_copy(mem_hbm.at[idx_v1], rows_v1, sem)
    cp0.wait()
    cp1.wait()

    for i in range(ntasks):
        rv = rows_v0 if i < ntasks // 2 else rows_v1
        roff = (i % (ntasks // 2)) * _K
        wv = [wgt_v[pl.ds(i * _K + g * 16, 16)] for g in range(_K // 16)]
        for c in range(_M // 16):
            acc = jnp.zeros((16,), jnp.float32)
            for j in range(_K):
                wj = wv[j // 16][j % 16]
                acc = acc + wj * rv[roff + j, pl.ds(c * 16, 16)]
            out_v[pl.ds(i * _M + c * 16, 16)] = acc
    pltpu.sync_copy(out_v, out_hbm.at[pl.ds(wid * ntasks * _M, ntasks * _M)])


def kernel(read_inputs, mem_state):
    keys = read_inputs[:, :_H * _M].reshape(_B, _H, _M)
    sraw = read_inputs[:, _H * _M:].reshape(_B, _H, 1)
    wgt, idx = pl.pallas_call(
        _score_body,
        grid=(_B // _NB,),
        in_specs=[
            pl.BlockSpec((_NB, _H, _M), lambda b: (b, 0, 0)),
            pl.BlockSpec((_NB, _H, 1), lambda b: (b, 0, 0)),
            pl.BlockSpec((_NB, _R, _M), lambda b: (b, 0, 0)),
        ],
        out_specs=[
            pl.BlockSpec((_NH, _K), lambda b: (b, 0)),
            pl.BlockSpec((_NH, _K), lambda b: (b, 0)),
        ],
        out_shape=[
            jax.ShapeDtypeStruct((_T, _K), jnp.float32),
            jax.ShapeDtypeStruct((_T, _K), jnp.int32),
        ],
    )(keys, sraw, mem_state)

    mesh = plsc.VectorSubcoreMesh(core_axis_name="c", subcore_axis_name="s")
    gather_sum = pl.kernel(
        _gather_body, mesh=mesh,
        out_type=jax.ShapeDtypeStruct((_T * _M,), jnp.float32),
        scratch_types=[
            pltpu.VMEM((128,), jnp.int32),
            pltpu.VMEM((128,), jnp.int32),
            pltpu.VMEM((128, _M), jnp.float32),
            pltpu.VMEM((128, _M), jnp.float32),
            pltpu.VMEM((256,), jnp.float32),
            pltpu.VMEM((8 * _M,), jnp.float32),
            pltpu.SemaphoreType.DMA,
        ],
    )
    out = gather_sum(mem_state.reshape(_B * _R, _M),
                     idx.reshape(_T * _K), wgt.reshape(_T * _K))
    return out.reshape(_B, _H * _M)
